# baseline (speedup 1.0000x reference)
import functools

import jax
import jax.numpy as jnp
from jax import lax
from jax.experimental import pallas as pl
from jax.experimental.pallas import tpu as pltpu
from jax.experimental.pallas import tpu_sc as plsc

BATCH = 16384
FEAT = 128
NUM_CORES = 2
NUM_SUBCORES = 16
NUM_WORKERS = NUM_CORES * NUM_SUBCORES
ROWS_PER_WORKER = BATCH // NUM_WORKERS
CHUNK = 64
NCHUNK = ROWS_PER_WORKER // CHUNK
PREF = 3
LANES = 16

_mesh = plsc.VectorSubcoreMesh(core_axis_name="c", subcore_axis_name="s")


@functools.partial(
    pl.kernel,
    mesh=_mesh,
    out_type=jax.ShapeDtypeStruct((BATCH, FEAT), jnp.float32),
    scratch_types=[
        pltpu.VMEM((ROWS_PER_WORKER,), jnp.int32),
        pltpu.VMEM((NCHUNK, CHUNK, FEAT), jnp.float32),
        pltpu.VMEM((PREF + 1, CHUNK, FEAT), jnp.float32),
        pltpu.SemaphoreType.DMA,
        pltpu.SemaphoreType.DMA,
        pltpu.SemaphoreType.DMA,
        pltpu.SemaphoreType.DMA,
    ],
)
def _film_sc(x_hbm, ids_hbm, gamma_hbm, beta_hbm, out_hbm,
             idx_v, g_v, x_v, sem_g, sem_b, sem_x, sem_o):
    wid = lax.axis_index("s") * NUM_CORES + lax.axis_index("c")
    base = wid * ROWS_PER_WORKER

    pltpu.sync_copy(ids_hbm.at[pl.ds(base, ROWS_PER_WORKER)], idx_v)

    def issue_g(c):
        return pltpu.async_copy(
            gamma_hbm.at[idx_v.at[pl.ds(c * CHUNK, CHUNK)]], g_v.at[c], sem_g)

    def issue_x(c):
        return pltpu.async_copy(
            x_hbm.at[pl.ds(base + c * CHUNK, CHUNK)], x_v.at[c % (PREF + 1)], sem_x)

    hg = [None] * NCHUNK
    hx = [None] * NCHUNK
    badd = [None] * NCHUNK
    wb = [None] * NCHUNK
    for c in range(PREF):
        hg[c] = issue_g(c)
        hx[c] = issue_x(c)

    for c in range(NCHUNK):
        hg[c].wait()
        hx[c].wait()
        if c + PREF < NCHUNK:
            hg[c + PREF] = issue_g(c + PREF)

        def row_body(r, carry):
            for j in range(FEAT // LANES):
                sl = pl.ds(j * LANES, LANES)
                g_v[c, r, sl] = g_v[c, r, sl] * x_v[c % (PREF + 1), r, sl]
            return carry

        lax.fori_loop(0, CHUNK, row_body, 0)
        badd[c] = pltpu.async_copy(beta_hbm.at[idx_v.at[pl.ds(c * CHUNK, CHUNK)]],
                                   g_v.at[c], sem_b, add=True)
        if c + PREF < NCHUNK:
            hx[c + PREF] = issue_x(c + PREF)
        if c >= 1:
            badd[c - 1].wait()
            wb[c - 1] = pltpu.async_copy(
                g_v.at[c - 1], out_hbm.at[pl.ds(base + (c - 1) * CHUNK, CHUNK)], sem_o)

    badd[NCHUNK - 1].wait()
    wb[NCHUNK - 1] = pltpu.async_copy(
        g_v.at[NCHUNK - 1],
        out_hbm.at[pl.ds(base + (NCHUNK - 1) * CHUNK, CHUNK)], sem_o)
    for h in wb:
        h.wait()


def kernel(x, domain_ids, gamma, beta):
    return _film_sc(x, domain_ids.astype(jnp.int32), gamma, beta)


# speedup vs baseline: 1.0383x; 1.0383x over previous
import functools

import jax
import jax.numpy as jnp
from jax import lax
from jax.experimental import pallas as pl
from jax.experimental.pallas import tpu as pltpu
from jax.experimental.pallas import tpu_sc as plsc

BATCH = 16384
FEAT = 128
NUM_CORES = 2
NUM_SUBCORES = 16
NUM_WORKERS = NUM_CORES * NUM_SUBCORES
ROWS_PER_WORKER = BATCH // NUM_WORKERS
CHUNK = 128
NCHUNK = ROWS_PER_WORKER // CHUNK
PREF = 2
LANES = 16

_mesh = plsc.VectorSubcoreMesh(core_axis_name="c", subcore_axis_name="s")


@functools.partial(
    pl.kernel,
    mesh=_mesh,
    out_type=jax.ShapeDtypeStruct((BATCH, FEAT), jnp.float32),
    scratch_types=[
        pltpu.VMEM((ROWS_PER_WORKER,), jnp.int32),
        pltpu.VMEM((NCHUNK, CHUNK, FEAT), jnp.float32),
        pltpu.VMEM((PREF + 1, CHUNK, FEAT), jnp.float32),
        pltpu.SemaphoreType.DMA,
        pltpu.SemaphoreType.DMA,
        pltpu.SemaphoreType.DMA,
        pltpu.SemaphoreType.DMA,
    ],
)
def _film_sc(x_hbm, ids_hbm, gamma_hbm, beta_hbm, out_hbm,
             idx_v, g_v, x_v, sem_g, sem_b, sem_x, sem_o):
    wid = lax.axis_index("s") * NUM_CORES + lax.axis_index("c")
    base = wid * ROWS_PER_WORKER

    pltpu.sync_copy(ids_hbm.at[pl.ds(base, ROWS_PER_WORKER)], idx_v)

    def issue_g(c):
        return pltpu.async_copy(
            gamma_hbm.at[idx_v.at[pl.ds(c * CHUNK, CHUNK)]], g_v.at[c], sem_g)

    def issue_x(c):
        return pltpu.async_copy(
            x_hbm.at[pl.ds(base + c * CHUNK, CHUNK)], x_v.at[c % (PREF + 1)], sem_x)

    hg = [None] * NCHUNK
    hx = [None] * NCHUNK
    badd = [None] * NCHUNK
    wb = [None] * NCHUNK
    for c in range(NCHUNK):
        hg[c] = issue_g(c)
    for c in range(PREF):
        hx[c] = issue_x(c)

    for c in range(NCHUNK):
        hg[c].wait()
        hx[c].wait()

        def row_body(r, carry):
            for j in range(FEAT // LANES):
                sl = pl.ds(j * LANES, LANES)
                g_v[c, r, sl] = g_v[c, r, sl] * x_v[c % (PREF + 1), r, sl]
            return carry

        lax.fori_loop(0, CHUNK, row_body, 0)
        badd[c] = pltpu.async_copy(beta_hbm.at[idx_v.at[pl.ds(c * CHUNK, CHUNK)]],
                                   g_v.at[c], sem_b, add=True)
        if c + PREF < NCHUNK:
            hx[c + PREF] = issue_x(c + PREF)
        if c >= 1:
            badd[c - 1].wait()
            wb[c - 1] = pltpu.async_copy(
                g_v.at[c - 1], out_hbm.at[pl.ds(base + (c - 1) * CHUNK, CHUNK)], sem_o)

    badd[NCHUNK - 1].wait()
    wb[NCHUNK - 1] = pltpu.async_copy(
        g_v.at[NCHUNK - 1],
        out_hbm.at[pl.ds(base + (NCHUNK - 1) * CHUNK, CHUNK)], sem_o)
    for h in wb:
        h.wait()


def kernel(x, domain_ids, gamma, beta):
    return _film_sc(x, domain_ids.astype(jnp.int32), gamma, beta)
